# Initial kernel scaffold; baseline (speedup 1.0000x reference)
import jax
import jax.numpy as jnp
from jax.experimental import pallas as pl


def kernel(x, embedding):
    raise NotImplementedError("write your pallas kernel here")


# speedup vs baseline: 2.0407x; 2.0407x over previous
import jax
import jax.numpy as jnp
from jax import lax
from jax.experimental import pallas as pl
from jax.experimental.pallas import tpu as pltpu
from jax.experimental.pallas import tpu_sc as plsc

_NUM_EMBEDDINGS = 1_000_000
_DIM = 64
_TP_SIZE = 4
_TP_RANK = 1
_PER_PART = _NUM_EMBEDDINGS // _TP_SIZE
_VSTART = _PER_PART * _TP_RANK
_VEND = _VSTART + _PER_PART

_B = 16384 * 50
_NC = 2
_NS = 16
_NW = _NC * _NS
_CHUNK = _B // _NW
_G = 128
_NG = _CHUNK // _G
_L = 16


def _body(x_hbm, tab_hbm, out_hbm, xin_v, lidx_v, rows_v, sem):
    wid = lax.axis_index("s") * _NC + lax.axis_index("c")
    base = wid * _CHUNK
    iota16 = lax.iota(jnp.int32, _L)
    zeros16 = jnp.zeros((_L,), jnp.float32)

    def group(g, carry):
        gbase = base + g * _G
        pltpu.sync_copy(x_hbm.at[pl.ds(gbase, _G)], xin_v)

        for i in range(_G // _L):
            xv = xin_v[pl.ds(i * _L, _L)]
            m = (xv >= _VSTART) & (xv < _VEND)
            lidx_v[pl.ds(i * _L, _L)] = jnp.where(m, xv - _VSTART, 0)

        pltpu.async_copy(tab_hbm.at[lidx_v], rows_v, sem).wait()

        for i in range(_G // _L):
            xv = xin_v[pl.ds(i * _L, _L)]
            mf = jnp.where((xv >= _VSTART) & (xv < _VEND), 1.0, 0.0)
            for j in range(_L):
                t = i * _L + j
                em = lax.gather(
                    mf, jnp.full((_L, 1), j, jnp.int32),
                    lax.GatherDimensionNumbers(
                        offset_dims=(), collapsed_slice_dims=(0,),
                        start_index_map=(0,)),
                    slice_sizes=(1,),
                    mode=lax.GatherScatterMode.PROMISE_IN_BOUNDS)
                for k in range(_DIM // _L):
                    rv = rows_v[t, pl.ds(k * _L, _L)]
                    rows_v[t, pl.ds(k * _L, _L)] = rv * em

        pltpu.sync_copy(rows_v, out_hbm.at[pl.ds(gbase, _G)])
        return carry

    lax.fori_loop(0, _NG, group, 0)


def kernel(x, embedding):
    xf = x.reshape(-1)
    mesh = plsc.VectorSubcoreMesh(core_axis_name="c", subcore_axis_name="s")
    f = pl.kernel(
        _body,
        out_type=jax.ShapeDtypeStruct((_B, _DIM), jnp.float32),
        mesh=mesh,
        compiler_params=pltpu.CompilerParams(use_tc_tiling_on_sc=False),
        scratch_types=[
            pltpu.VMEM((_G,), jnp.int32),
            pltpu.VMEM((_G,), jnp.int32),
            pltpu.VMEM((_G, _DIM), jnp.float32),
            pltpu.SemaphoreType.DMA,
        ],
    )
    out = f(xf, embedding)
    return out.reshape(x.shape[0], x.shape[1], _DIM)


# speedup vs baseline: 2.0413x; 1.0003x over previous
import jax
import jax.numpy as jnp
from jax import lax
from jax.experimental import pallas as pl
from jax.experimental.pallas import tpu as pltpu
from jax.experimental.pallas import tpu_sc as plsc

_NUM_EMBEDDINGS = 1_000_000
_DIM = 64
_TP_SIZE = 4
_TP_RANK = 1
_PER_PART = _NUM_EMBEDDINGS // _TP_SIZE
_VSTART = _PER_PART * _TP_RANK
_VEND = _VSTART + _PER_PART

_B = 16384 * 50
_NC = 2
_NS = 16
_NW = _NC * _NS
_CHUNK = _B // _NW
_G = 128
_S = 512
_NGS = _S // _G
_NSUP = _CHUNK // _S
_NP = _NSUP // 2
_L = 16

_BCAST_DNUMS = lax.GatherDimensionNumbers(
    offset_dims=(), collapsed_slice_dims=(0,), start_index_map=(0,))


def _body(x_hbm, tab_hbm, out_hbm,
          xin_a, xin_b, lidx_a, lidx_b, rows_a, rows_b,
          sem_ga, sem_gb, sem_wa, sem_wb):
    wid = lax.axis_index("s") * _NC + lax.axis_index("c")
    base = wid * _CHUNK

    def fire(sc, xin_v, lidx_v, rows_v, sem_g):
        sbase = base + sc * _S
        pltpu.sync_copy(x_hbm.at[pl.ds(sbase, _S)], xin_v)
        for g in range(_NGS):
            def mk(i, c, g=g):
                xv = xin_v[pl.ds(g * _G + i * _L, _L)]
                m = (xv >= _VSTART) & (xv < _VEND)
                lidx_v[g, pl.ds(i * _L, _L)] = jnp.where(m, xv - _VSTART, 0)
                return c
            lax.fori_loop(0, _G // _L, mk, 0)
            pltpu.async_copy(tab_hbm.at[lidx_v.at[g]],
                             rows_v.at[pl.ds(g * _G, _G)], sem_g)

    def drain(lidx_v, rows_v, sem_g):
        for g in range(_NGS):
            pltpu.make_async_copy(tab_hbm.at[lidx_v.at[g]],
                                  rows_v.at[pl.ds(g * _G, _G)], sem_g).wait()

    def process(xin_v, rows_v):
        def grp(i, c):
            xv = xin_v[pl.ds(i * _L, _L)]
            mf = jnp.where((xv >= _VSTART) & (xv < _VEND), 1.0, 0.0)
            for j in range(_L):
                em = lax.gather(mf, jnp.full((_L, 1), j, jnp.int32),
                                _BCAST_DNUMS, slice_sizes=(1,),
                                mode=lax.GatherScatterMode.PROMISE_IN_BOUNDS)
                def mul(t, c2, em=em):
                    for k in range(_DIM // _L):
                        rows_v[t, pl.ds(k * _L, _L)] = (
                            rows_v[t, pl.ds(k * _L, _L)] * em)
                    return c2
                mul(i * _L + j, 0)
            return c
        lax.fori_loop(0, _S // _L, grp, 0)

    def writeback(sc, rows_v, sem_w):
        sbase = base + sc * _S
        pltpu.async_copy(rows_v, out_hbm.at[pl.ds(sbase, _S)], sem_w)

    def wait_wb(rows_v, sem_w):
        pltpu.make_async_copy(rows_v, out_hbm.at[pl.ds(base, _S)], sem_w).wait()

    fire(0, xin_a, lidx_a, rows_a, sem_ga)

    def pair(p, carry):
        sc0 = 2 * p
        @pl.when(p > 0)
        def _():
            wait_wb(rows_b, sem_wb)
        fire(sc0 + 1, xin_b, lidx_b, rows_b, sem_gb)
        drain(lidx_a, rows_a, sem_ga)
        process(xin_a, rows_a)
        writeback(sc0, rows_a, sem_wa)
        @pl.when(p < _NP - 1)
        def _():
            wait_wb(rows_a, sem_wa)
            fire(sc0 + 2, xin_a, lidx_a, rows_a, sem_ga)
        drain(lidx_b, rows_b, sem_gb)
        process(xin_b, rows_b)
        writeback(sc0 + 1, rows_b, sem_wb)
        return carry

    lax.fori_loop(0, _NP, pair, 0)
    wait_wb(rows_a, sem_wa)
    wait_wb(rows_b, sem_wb)


def kernel(x, embedding):
    xf = x.reshape(-1)
    mesh = plsc.VectorSubcoreMesh(core_axis_name="c", subcore_axis_name="s")
    f = pl.kernel(
        _body,
        out_type=jax.ShapeDtypeStruct((_B, _DIM), jnp.float32),
        mesh=mesh,
        compiler_params=pltpu.CompilerParams(use_tc_tiling_on_sc=False),
        scratch_types=[
            pltpu.VMEM((_S,), jnp.int32),
            pltpu.VMEM((_S,), jnp.int32),
            pltpu.VMEM((_NGS, _G), jnp.int32),
            pltpu.VMEM((_NGS, _G), jnp.int32),
            pltpu.VMEM((_S, _DIM), jnp.float32),
            pltpu.VMEM((_S, _DIM), jnp.float32),
            pltpu.SemaphoreType.DMA,
            pltpu.SemaphoreType.DMA,
            pltpu.SemaphoreType.DMA,
            pltpu.SemaphoreType.DMA,
        ],
    )
    out = f(xf, embedding)
    return out.reshape(x.shape[0], x.shape[1], _DIM)


# speedup vs baseline: 21.9074x; 10.7321x over previous
import jax
import jax.numpy as jnp
from jax import lax
from jax.experimental import pallas as pl
from jax.experimental.pallas import tpu as pltpu
from jax.experimental.pallas import tpu_sc as plsc

_NUM_EMBEDDINGS = 1_000_000
_DIM = 64
_TP_SIZE = 4
_TP_RANK = 1
_PER_PART = _NUM_EMBEDDINGS // _TP_SIZE
_VSTART = _PER_PART * _TP_RANK
_VEND = _VSTART + _PER_PART

_B = 16384 * 50
_NC = 2
_NS = 16
_NW = _NC * _NS
_CHUNK = _B // _NW
_G = 128
_S = 512
_NGS = _S // _G
_NSUP = _CHUNK // _S
_NP = _NSUP // 2
_L = 16

_BCAST_DNUMS = lax.GatherDimensionNumbers(
    offset_dims=(), collapsed_slice_dims=(0,), start_index_map=(0,))


def _body(x_hbm, tab_hbm, out_hbm,
          xin_a, xin_b, lidx_a, lidx_b, rows_a, rows_b,
          sem_ga, sem_gb, sem_wa, sem_wb):
    wid = lax.axis_index("s") * _NC + lax.axis_index("c")
    base = wid * _CHUNK

    def fire(sc, xin_v, lidx_v, rows_v, sem_g):
        sbase = base + sc * _S
        pltpu.sync_copy(x_hbm.at[pl.ds(sbase, _S)], xin_v)
        for g in range(_NGS):
            def mk(i, c, g=g):
                xv = xin_v[pl.ds(g * _G + i * _L, _L)]
                m = (xv >= _VSTART) & (xv < _VEND)
                lidx_v[g, pl.ds(i * _L, _L)] = jnp.where(
                    m, xv - _VSTART, xv & 0x1FFFF)
                return c
            lax.fori_loop(0, _G // _L, mk, 0)
            pltpu.async_copy(tab_hbm.at[lidx_v.at[g]],
                             rows_v.at[pl.ds(g * _G, _G)], sem_g)

    def drain(lidx_v, rows_v, sem_g):
        for g in range(_NGS):
            pltpu.make_async_copy(tab_hbm.at[lidx_v.at[g]],
                                  rows_v.at[pl.ds(g * _G, _G)], sem_g).wait()

    def process(xin_v, rows_v):
        def grp(i, c):
            xv = xin_v[pl.ds(i * _L, _L)]
            mf = jnp.where((xv >= _VSTART) & (xv < _VEND), 1.0, 0.0)
            for j in range(_L):
                em = lax.gather(mf, jnp.full((_L, 1), j, jnp.int32),
                                _BCAST_DNUMS, slice_sizes=(1,),
                                mode=lax.GatherScatterMode.PROMISE_IN_BOUNDS)
                def mul(t, c2, em=em):
                    for k in range(_DIM // _L):
                        rows_v[t, pl.ds(k * _L, _L)] = (
                            rows_v[t, pl.ds(k * _L, _L)] * em)
                    return c2
                mul(i * _L + j, 0)
            return c
        lax.fori_loop(0, _S // _L, grp, 0)

    def writeback(sc, rows_v, sem_w):
        sbase = base + sc * _S
        pltpu.async_copy(rows_v, out_hbm.at[pl.ds(sbase, _S)], sem_w)

    def wait_wb(rows_v, sem_w):
        pltpu.make_async_copy(rows_v, out_hbm.at[pl.ds(base, _S)], sem_w).wait()

    fire(0, xin_a, lidx_a, rows_a, sem_ga)

    def pair(p, carry):
        sc0 = 2 * p
        @pl.when(p > 0)
        def _():
            wait_wb(rows_b, sem_wb)
        fire(sc0 + 1, xin_b, lidx_b, rows_b, sem_gb)
        drain(lidx_a, rows_a, sem_ga)
        process(xin_a, rows_a)
        writeback(sc0, rows_a, sem_wa)
        @pl.when(p < _NP - 1)
        def _():
            wait_wb(rows_a, sem_wa)
            fire(sc0 + 2, xin_a, lidx_a, rows_a, sem_ga)
        drain(lidx_b, rows_b, sem_gb)
        process(xin_b, rows_b)
        writeback(sc0 + 1, rows_b, sem_wb)
        return carry

    lax.fori_loop(0, _NP, pair, 0)
    wait_wb(rows_a, sem_wa)
    wait_wb(rows_b, sem_wb)


def kernel(x, embedding):
    xf = x.reshape(-1)
    mesh = plsc.VectorSubcoreMesh(core_axis_name="c", subcore_axis_name="s")
    f = pl.kernel(
        _body,
        out_type=jax.ShapeDtypeStruct((_B, _DIM), jnp.float32),
        mesh=mesh,
        compiler_params=pltpu.CompilerParams(use_tc_tiling_on_sc=False),
        scratch_types=[
            pltpu.VMEM((_S,), jnp.int32),
            pltpu.VMEM((_S,), jnp.int32),
            pltpu.VMEM((_NGS, _G), jnp.int32),
            pltpu.VMEM((_NGS, _G), jnp.int32),
            pltpu.VMEM((_S, _DIM), jnp.float32),
            pltpu.VMEM((_S, _DIM), jnp.float32),
            pltpu.SemaphoreType.DMA,
            pltpu.SemaphoreType.DMA,
            pltpu.SemaphoreType.DMA,
            pltpu.SemaphoreType.DMA,
        ],
    )
    out = f(xf, embedding)
    return out.reshape(x.shape[0], x.shape[1], _DIM)


# speedup vs baseline: 28.3552x; 1.2943x over previous
import jax
import jax.numpy as jnp
from jax import lax
from jax.experimental import pallas as pl
from jax.experimental.pallas import tpu as pltpu
from jax.experimental.pallas import tpu_sc as plsc

_NUM_EMBEDDINGS = 1_000_000
_DIM = 64
_TP_SIZE = 4
_TP_RANK = 1
_PER_PART = _NUM_EMBEDDINGS // _TP_SIZE
_VSTART = _PER_PART * _TP_RANK
_VEND = _VSTART + _PER_PART

_ZPAD = 4096
_ZMASK = _ZPAD - 1

_B = 16384 * 50
_NC = 2
_NS = 16
_NW = _NC * _NS
_CHUNK = _B // _NW
_G = 128
_S = 512
_NGS = _S // _G
_NSUP = _CHUNK // _S
_NP = _NSUP // 2
_L = 16


def _body(x_hbm, tab_hbm, out_hbm,
          xin_a, xin_b, lidx_a, lidx_b, rows_a, rows_b,
          sem_ga, sem_gb, sem_wa, sem_wb):
    wid = lax.axis_index("s") * _NC + lax.axis_index("c")
    base = wid * _CHUNK

    def fire(sc, xin_v, lidx_v, rows_v, sem_g):
        sbase = base + sc * _S
        pltpu.sync_copy(x_hbm.at[pl.ds(sbase, _S)], xin_v)
        for g in range(_NGS):
            def mk(i, c, g=g):
                xv = xin_v[pl.ds(g * _G + i * _L, _L)]
                m = (xv >= _VSTART) & (xv < _VEND)
                lidx_v[g, pl.ds(i * _L, _L)] = jnp.where(
                    m, xv - _VSTART, _PER_PART + (xv & _ZMASK))
                return c
            lax.fori_loop(0, _G // _L, mk, 0)
            pltpu.async_copy(tab_hbm.at[lidx_v.at[g]],
                             rows_v.at[pl.ds(g * _G, _G)], sem_g)

    def drain(lidx_v, rows_v, sem_g):
        for g in range(_NGS):
            pltpu.make_async_copy(tab_hbm.at[lidx_v.at[g]],
                                  rows_v.at[pl.ds(g * _G, _G)], sem_g).wait()

    def writeback(sc, rows_v, sem_w):
        sbase = base + sc * _S
        pltpu.async_copy(rows_v, out_hbm.at[pl.ds(sbase, _S)], sem_w)

    def wait_wb(rows_v, sem_w):
        pltpu.make_async_copy(rows_v, out_hbm.at[pl.ds(base, _S)], sem_w).wait()

    fire(0, xin_a, lidx_a, rows_a, sem_ga)

    def pair(p, carry):
        sc0 = 2 * p
        @pl.when(p > 0)
        def _():
            wait_wb(rows_b, sem_wb)
        fire(sc0 + 1, xin_b, lidx_b, rows_b, sem_gb)
        drain(lidx_a, rows_a, sem_ga)
        writeback(sc0, rows_a, sem_wa)
        @pl.when(p < _NP - 1)
        def _():
            wait_wb(rows_a, sem_wa)
            fire(sc0 + 2, xin_a, lidx_a, rows_a, sem_ga)
        drain(lidx_b, rows_b, sem_gb)
        writeback(sc0 + 1, rows_b, sem_wb)
        return carry

    lax.fori_loop(0, _NP, pair, 0)
    wait_wb(rows_a, sem_wa)
    wait_wb(rows_b, sem_wb)


def kernel(x, embedding):
    xf = x.reshape(-1)
    tab = jnp.concatenate(
        [embedding, jnp.zeros((_ZPAD, _DIM), jnp.float32)], axis=0)
    mesh = plsc.VectorSubcoreMesh(core_axis_name="c", subcore_axis_name="s")
    f = pl.kernel(
        _body,
        out_type=jax.ShapeDtypeStruct((_B, _DIM), jnp.float32),
        mesh=mesh,
        compiler_params=pltpu.CompilerParams(use_tc_tiling_on_sc=False),
        scratch_types=[
            pltpu.VMEM((_S,), jnp.int32),
            pltpu.VMEM((_S,), jnp.int32),
            pltpu.VMEM((_NGS, _G), jnp.int32),
            pltpu.VMEM((_NGS, _G), jnp.int32),
            pltpu.VMEM((_S, _DIM), jnp.float32),
            pltpu.VMEM((_S, _DIM), jnp.float32),
            pltpu.SemaphoreType.DMA,
            pltpu.SemaphoreType.DMA,
            pltpu.SemaphoreType.DMA,
            pltpu.SemaphoreType.DMA,
        ],
    )
    out = f(xf, tab)
    return out.reshape(x.shape[0], x.shape[1], _DIM)
